# baseline (speedup 1.0000x reference)
import jax
import jax.numpy as jnp
from jax import lax
from jax.experimental import pallas as pl
from jax.experimental.pallas import tpu as pltpu
from jax.experimental.pallas import tpu_sc as plsc

TIME_STEPS = 1000
EMBED_DIM = 128
BATCH = 16384

_info = plsc.get_sparse_core_info()
_NC, _NS = _info.num_cores, _info.num_subcores
_NW = _NC * _NS
_BPW = BATCH // _NW
_ROWS_PER_STAGER = 64
_TAIL_BASE = 15 * _ROWS_PER_STAGER
_TAIL_ROWS = TIME_STEPS - _TAIL_BASE


def _gather_body(table_hbm, idx_hbm, out_hbm, tab_sh, idx_v, rows_v, isem, gsem):
    cid = lax.axis_index("c")
    sid = lax.axis_index("s")
    wid = sid * _NC + cid
    base = wid * _BPW
    idx_cp = pltpu.async_copy(idx_hbm.at[pl.ds(base, _BPW)], idx_v, isem)

    @pl.when(sid < 15)
    def _stage():
        r0 = sid * _ROWS_PER_STAGER
        pltpu.sync_copy(
            table_hbm.at[pl.ds(r0, _ROWS_PER_STAGER)],
            tab_sh.at[pl.ds(r0, _ROWS_PER_STAGER)],
        )

    @pl.when(sid == 15)
    def _stage_tail():
        pltpu.sync_copy(
            table_hbm.at[pl.ds(_TAIL_BASE, _TAIL_ROWS)],
            tab_sh.at[pl.ds(_TAIL_BASE, _TAIL_ROWS)],
        )

    plsc.subcore_barrier()
    idx_cp.wait()
    pltpu.async_copy(tab_sh.at[idx_v], rows_v, gsem).wait()
    pltpu.sync_copy(rows_v, out_hbm.at[pl.ds(base, _BPW)])


_mesh = plsc.VectorSubcoreMesh(core_axis_name="c", subcore_axis_name="s")


@jax.jit
def _gather(table, idx):
    return pl.kernel(
        _gather_body,
        mesh=_mesh,
        out_type=jax.ShapeDtypeStruct((BATCH, EMBED_DIM), jnp.float32),
        scratch_types=[
            pltpu.VMEM_SHARED((TIME_STEPS, EMBED_DIM), jnp.float32),
            pltpu.VMEM((_BPW,), jnp.int32),
            pltpu.VMEM((_BPW, EMBED_DIM), jnp.float32),
            pltpu.SemaphoreType.DMA,
            pltpu.SemaphoreType.DMA,
        ],
    )(table, idx)


def kernel(x, t, embeddings):
    out = _gather(embeddings, t.astype(jnp.int32))
    return out[:, :, None, None]
